# baseline (speedup 1.0000x reference)
import functools

import jax
import jax.numpy as jnp
from jax import lax
from jax.experimental import pallas as pl
from jax.experimental.pallas import tpu as pltpu
from jax.experimental.pallas import tpu_sc as plsc

B, H, E, O = 16384, 50, 32, 64
N = B * H
NC, NS = 2, 16
NW = NC * NS
GCHUNK = 128
FOLD = 128 // E
OF = O * FOLD
NSPLIT = 2
NH = N // NSPLIT


def _sc_gather(idx2d, table, n_rows):
    per_w = n_rows // NW
    chunk = 1280
    ng = chunk // GCHUNK
    nchunks = per_w // chunk
    cf = chunk // FOLD
    mesh = plsc.VectorSubcoreMesh(core_axis_name="c", subcore_axis_name="s")

    @functools.partial(
        pl.kernel,
        mesh=mesh,
        out_type=jax.ShapeDtypeStruct((n_rows // FOLD, 128), jnp.float32),
        scratch_types=[
            pltpu.VMEM((ng, GCHUNK), jnp.int32),
            pltpu.VMEM((chunk, E), jnp.float32),
            pltpu.VMEM((cf, 128), jnp.float32),
            pltpu.SemaphoreType.DMA,
        ],
        compiler_params=pltpu.CompilerParams(use_tc_tiling_on_sc=False),
    )
    def k(idx_hbm, table_hbm, out_hbm, idx_v, rows_v, rows_f, sem):
        wid = lax.axis_index("s") * NC + lax.axis_index("c")
        base = wid * per_w

        def body(i, carry):
            off = base + i * chunk
            pltpu.sync_copy(
                idx_hbm.at[pl.ds(pl.multiple_of(off // GCHUNK, 2), ng)], idx_v
            )
            copies = [
                pltpu.async_copy(
                    table_hbm.at[idx_v.at[j]],
                    rows_v.at[pl.ds(j * GCHUNK, GCHUNK)],
                    sem,
                )
                for j in range(ng)
            ]
            for cp in copies:
                cp.wait()

            def fold_body(r, c2):
                for u in range(8):
                    v = rows_v[FOLD * r + u // 2, pl.ds(16 * (u % 2), 16)]
                    rows_f[r, pl.ds(16 * u, 16)] = v
                return c2

            lax.fori_loop(0, cf, fold_body, 0)

            pltpu.sync_copy(
                rows_f,
                out_hbm.at[pl.ds(pl.multiple_of(off // FOLD, 8), cf)],
            )
            return carry

        lax.fori_loop(0, nchunks, body, 0)

    return k(idx2d, table)


def _tc_linear_relu(xf, wd, bf, n_fold):
    blk = 2048
    nblk = n_fold // blk

    def body(x_hbm, w_ref, b_ref, o_ref, xv, sem):
        i = pl.program_id(0)

        @pl.when(i == 0)
        def _():
            pltpu.make_async_copy(x_hbm.at[pl.ds(0, blk)], xv.at[0], sem).start()

        @pl.when(i + 1 < nblk)
        def _():
            pltpu.make_async_copy(
                x_hbm.at[pl.ds((i + 1) * blk, blk)], xv.at[(i + 1) % 2], sem
            ).start()

        pltpu.make_async_copy(x_hbm.at[pl.ds(i * blk, blk)], xv.at[i % 2], sem).wait()
        acc = jnp.dot(xv[i % 2], w_ref[...], preferred_element_type=jnp.float32)
        o_ref[...] = jnp.maximum(acc + b_ref[...], 0.0)

    return pl.pallas_call(
        body,
        grid=(nblk,),
        in_specs=[
            pl.BlockSpec(memory_space=pltpu.MemorySpace.HBM),
            pl.BlockSpec((128, OF), lambda i: (0, 0)),
            pl.BlockSpec((1, OF), lambda i: (0, 0)),
        ],
        out_specs=pl.BlockSpec((blk, OF), lambda i: (i, 0)),
        out_shape=jax.ShapeDtypeStruct((n_fold, OF), jnp.float32),
        scratch_shapes=[
            pltpu.VMEM((2, blk, 128), jnp.float32),
            pltpu.SemaphoreType.DMA,
        ],
    )(xf, wd, bf)


def kernel(category, table, W, b):
    idx2d = category.astype(jnp.int32).reshape(N // GCHUNK, GCHUNK)
    wd = jnp.kron(jnp.eye(FOLD, dtype=jnp.float32), W.T)
    bf = jnp.tile(b, FOLD).reshape(1, OF)
    rows_per = NH // GCHUNK
    outs = []
    for s in range(NSPLIT):
        xf = _sc_gather(idx2d[s * rows_per:(s + 1) * rows_per], table, NH)
        outs.append(_tc_linear_relu(xf, wd, bf, NH // FOLD))
    out = jnp.concatenate(outs, axis=0)
    return out.reshape(B, H, O)


# speedup vs baseline: 1.0681x; 1.0681x over previous
import functools

import jax
import jax.numpy as jnp
from jax import lax
from jax.experimental import pallas as pl
from jax.experimental.pallas import tpu as pltpu
from jax.experimental.pallas import tpu_sc as plsc

B, H, E, O = 16384, 50, 32, 64
N = B * H
NC, NS = 2, 16
NW = NC * NS
GCHUNK = 128
FOLD = 128 // E
OF = O * FOLD
NSPLIT = 1
NH = N // NSPLIT


def _sc_gather(idx2d, table, n_rows):
    per_w = n_rows // NW
    chunk = 1024
    ng = chunk // GCHUNK
    nchunks = per_w // chunk
    cf = chunk // FOLD
    mesh = plsc.VectorSubcoreMesh(core_axis_name="c", subcore_axis_name="s")

    @functools.partial(
        pl.kernel,
        mesh=mesh,
        out_type=jax.ShapeDtypeStruct((n_rows // FOLD, 128), jnp.float32),
        scratch_types=[
            pltpu.VMEM((ng, GCHUNK), jnp.int32),
            pltpu.VMEM((chunk, E), jnp.float32),
            pltpu.VMEM((cf, 128), jnp.float32),
            pltpu.SemaphoreType.DMA,
        ],
        compiler_params=pltpu.CompilerParams(use_tc_tiling_on_sc=False),
    )
    def k(idx_hbm, table_hbm, out_hbm, idx_v, rows_v, rows_f, sem):
        wid = lax.axis_index("s") * NC + lax.axis_index("c")
        base = wid * per_w

        def body(i, carry):
            off = base + i * chunk
            pltpu.sync_copy(
                idx_hbm.at[pl.ds(pl.multiple_of(off // GCHUNK, 2), ng)], idx_v
            )
            copies = [
                pltpu.async_copy(
                    table_hbm.at[idx_v.at[j]],
                    rows_v.at[pl.ds(j * GCHUNK, GCHUNK)],
                    sem,
                )
                for j in range(ng)
            ]
            for cp in copies:
                cp.wait()

            def fold_body(r, c2):
                for u in range(8):
                    v = rows_v[FOLD * r + u // 2, pl.ds(16 * (u % 2), 16)]
                    rows_f[r, pl.ds(16 * u, 16)] = v
                return c2

            lax.fori_loop(0, cf, fold_body, 0)

            pltpu.sync_copy(
                rows_f,
                out_hbm.at[pl.ds(pl.multiple_of(off // FOLD, 8), cf)],
            )
            return carry

        lax.fori_loop(0, nchunks, body, 0)

    return k(idx2d, table)


def _tc_linear_relu(xf, wd, bf, n_fold):
    blk = 2048
    nblk = n_fold // blk

    def body(x_hbm, w_ref, b_ref, o_ref, xv, sem):
        i = pl.program_id(0)

        @pl.when(i == 0)
        def _():
            pltpu.make_async_copy(x_hbm.at[pl.ds(0, blk)], xv.at[0], sem).start()

        @pl.when(i + 1 < nblk)
        def _():
            pltpu.make_async_copy(
                x_hbm.at[pl.ds((i + 1) * blk, blk)], xv.at[(i + 1) % 2], sem
            ).start()

        pltpu.make_async_copy(x_hbm.at[pl.ds(i * blk, blk)], xv.at[i % 2], sem).wait()
        acc = jnp.dot(xv[i % 2], w_ref[...], preferred_element_type=jnp.float32)
        o_ref[...] = jnp.maximum(acc + b_ref[...], 0.0)

    return pl.pallas_call(
        body,
        grid=(nblk,),
        in_specs=[
            pl.BlockSpec(memory_space=pltpu.MemorySpace.HBM),
            pl.BlockSpec((128, OF), lambda i: (0, 0)),
            pl.BlockSpec((1, OF), lambda i: (0, 0)),
        ],
        out_specs=pl.BlockSpec((blk, OF), lambda i: (i, 0)),
        out_shape=jax.ShapeDtypeStruct((n_fold, OF), jnp.float32),
        scratch_shapes=[
            pltpu.VMEM((2, blk, 128), jnp.float32),
            pltpu.SemaphoreType.DMA,
        ],
    )(xf, wd, bf)


def kernel(category, table, W, b):
    idx2d = category.astype(jnp.int32).reshape(N // GCHUNK, GCHUNK)
    wd = jnp.kron(jnp.eye(FOLD, dtype=jnp.float32), W.T)
    bf = jnp.tile(b, FOLD).reshape(1, OF)
    rows_per = NH // GCHUNK
    outs = []
    for s in range(NSPLIT):
        xf = _sc_gather(idx2d[s * rows_per:(s + 1) * rows_per], table, NH)
        outs.append(_tc_linear_relu(xf, wd, bf, NH // FOLD))
    out = jnp.concatenate(outs, axis=0)
    return out.reshape(B, H, O)
